# Initial kernel scaffold; baseline (speedup 1.0000x reference)
import jax
import jax.numpy as jnp
from jax.experimental import pallas as pl


def kernel(from_s, from_v, to_s, to_v, edge_index, from_frame, to_frame, from_pos, to_pos, Wfs, Wts, Wfv, Wtv, Wattn):
    raise NotImplementedError("write your pallas kernel here")


# speedup vs baseline: 21.8177x; 21.8177x over previous
import functools

import jax
import jax.numpy as jnp
import numpy as np
from jax import lax
from jax.experimental import pallas as pl
from jax.experimental.pallas import tpu as pltpu
import jax.experimental.pallas.tpu_sc as plsc

N = 10000
E = 320000
NP = 10240
NW = 32
EPW = E // NW
C = 80
NCH = EPW // C
L = 16
SL = NP // 16
BN = 400

_S_ALL = np.zeros((12, 36), np.float32)
_T_ALL = np.zeros((9, 36), np.float32)
for _k in range(3):
    for _h in range(4):
        for _j in range(3):
            _S_ALL[3 * _h + _k, 12 * _k + 3 * _h + _j] = 1.0
            _T_ALL[3 * _k + _j, 12 * _k + 3 * _h + _j] = 1.0
_K3 = np.concatenate([np.eye(12, dtype=np.float32)] * 3, axis=0)


def _silu(x):
    return x * (1.0 / (1.0 + jnp.exp(-x)))


def _node_tc(s_ref, v_ref, f_ref, p_ref, WT_ref, W1_ref, ST_ref, K3_ref, wp_ref,
             tab_ref):
    s = s_ref[...]
    hs = jnp.dot(s, WT_ref[0], preferred_element_type=jnp.float32)
    a_s = jnp.sum(_silu(hs) * wp_ref[0, :, 0:16], axis=1, keepdims=True)
    P = jnp.dot(v_ref[...], W1_ref[0], preferred_element_type=jnp.float32)
    F = f_ref[...]
    PS = jnp.dot(P, ST_ref[0], preferred_element_type=jnp.float32)
    FT = jnp.dot(F, ST_ref[1, 0:9, :], preferred_element_type=jnp.float32)
    G = jnp.dot(PS * FT, K3_ref[...], preferred_element_type=jnp.float32)
    a_g = jnp.sum(_silu(G) * wp_ref[0, :, 16:28], axis=1, keepdims=True)
    a = a_s + a_g
    pad = jnp.zeros((BN, 3), jnp.float32)
    tab_ref[...] = jnp.concatenate([a, p_ref[...], F, pad], axis=1)


def _node_tables(s_in, v_in, f_in, p_in, WT, W1, ST, K3, wp):
    nb = (2 * N) // BN
    side = lambda b: b // (N // BN)
    return pl.pallas_call(
        _node_tc,
        grid=(nb,),
        in_specs=[
            pl.BlockSpec((BN, 128), lambda b: (b, 0)),
            pl.BlockSpec((BN, 48), lambda b: (b, 0)),
            pl.BlockSpec((BN, 9), lambda b: (b, 0)),
            pl.BlockSpec((BN, 3), lambda b: (b, 0)),
            pl.BlockSpec((1, 128, 16), lambda b: (side(b), 0, 0)),
            pl.BlockSpec((1, 48, 12), lambda b: (side(b), 0, 0)),
            pl.BlockSpec((2, 12, 36), lambda b: (0, 0, 0)),
            pl.BlockSpec((36, 12), lambda b: (0, 0)),
            pl.BlockSpec((1, 1, 32), lambda b: (side(b), 0, 0)),
        ],
        out_specs=pl.BlockSpec((BN, 16), lambda b: (b, 0)),
        out_shape=jax.ShapeDtypeStruct((2 * N, 16), jnp.float32),
    )(s_in, v_in, f_in, p_in, WT, W1, ST, K3, wp)


_MESH = plsc.VectorSubcoreMesh(core_axis_name="c", subcore_axis_name="s")


@functools.partial(
    pl.kernel,
    out_type=(jax.ShapeDtypeStruct((E,), jnp.float32),
              jax.ShapeDtypeStruct((NW * L,), jnp.float32)),
    mesh=_MESH,
    compiler_params=pltpu.CompilerParams(needs_layout_passes=False, use_tc_tiling_on_sc=False),
    scratch_types=[
        pltpu.VMEM((C,), jnp.int32),
        pltpu.VMEM((C,), jnp.int32),
        pltpu.VMEM((C, L), jnp.float32),
        pltpu.VMEM((C, L), jnp.float32),
        pltpu.VMEM((C,), jnp.float32),
        pltpu.VMEM((L,), jnp.float32),
        pltpu.VMEM((L,), jnp.float32),
        pltpu.VMEM((6 * L,), jnp.float32),
        pltpu.VMEM((7, C), jnp.float32),
        pltpu.SemaphoreType.DMA,
        pltpu.SemaphoreType.DMA,
    ],
)
def _raw_sc(tab, i0, i1p, wv, raw_out, mx_out,
            idx0_v, idx1_v, fr_v, tr_v, raw_v, wv_v, mx_v, ws_v, arg_v, sem, sem2):
    cid = lax.axis_index("c")
    sid = lax.axis_index("s")
    wid = sid * 2 + cid
    ebase = wid * EPW
    pltpu.sync_copy(wv, wv_v)
    for k in range(6):
        ws_v[pl.ds(k * L, L)] = plsc.load_gather(wv_v, [jnp.full((L,), k + 1, jnp.int32)])

    def chunk(c, mx):
        base = pl.multiple_of(ebase + c * C, 8)
        pltpu.sync_copy(i0.at[pl.ds(base, C)], idx0_v)
        pltpu.sync_copy(i1p.at[pl.ds(base, C)], idx1_v)
        pltpu.async_copy(tab.at[idx0_v], fr_v, sem).wait()
        pltpu.async_copy(tab.at[idx1_v], tr_v, sem2).wait()
        for g in range(C // L):
            rows = lax.iota(jnp.int32, L) + g * L
            sl = pl.ds(g * L, L)

            def colf(j):
                return plsc.load_gather(fr_v, [rows, jnp.full((L,), j, jnp.int32)])

            def colt(j):
                return plsc.load_gather(tr_v, [rows, jnp.full((L,), j, jnp.int32)])

            arg_v[6, sl] = colf(0) + colt(0)
            dx = colt(1) - colf(1)
            dy = colt(2) - colf(2)
            dz = colt(3) - colf(3)
            for j in range(3):
                arg_v[j, sl] = dx * colf(4 + j) + dy * colf(7 + j) + dz * colf(10 + j)
            for j in range(3):
                arg_v[3 + j, sl] = -(dx * colt(4 + j) + dy * colt(7 + j) + dz * colt(10 + j))
        for g in range(C // L):
            sl = pl.ds(g * L, L)
            acc = arg_v[6, sl]
            for j in range(6):
                cj = arg_v[j, sl]
                acc = acc + ws_v[pl.ds(j * L, L)] * (cj * (1.0 / (1.0 + jnp.exp(-cj))))
            raw_v[sl] = acc
            mx = jnp.maximum(mx, acc)
        pltpu.sync_copy(raw_v, raw_out.at[pl.ds(base, C)])
        return mx

    mx = lax.fori_loop(0, NCH, chunk, jnp.full((L,), -1e30, jnp.float32))
    mx_v[...] = mx
    pltpu.sync_copy(mx_v, mx_out.at[pl.ds(wid * L, L)])


@functools.partial(
    pl.kernel,
    out_type=(jax.ShapeDtypeStruct((E,), jnp.float32),
              jax.ShapeDtypeStruct((2, NP), jnp.float32)),
    mesh=_MESH,
    compiler_params=pltpu.CompilerParams(needs_layout_passes=False, use_tc_tiling_on_sc=False),
    scratch_types=[
        pltpu.VMEM((NW * L,), jnp.float32),
        pltpu.VMEM((C,), jnp.int32),
        pltpu.VMEM((C,), jnp.float32),
        pltpu.VMEM((C,), jnp.float32),
        pltpu.VMEM((NP,), jnp.float32),
        pltpu.VMEM((SL,), jnp.float32),
        pltpu.VMEM((SL,), jnp.float32),
        pltpu.VMEM_SHARED((16, NP), jnp.float32),
        pltpu.SemaphoreType.DMA,
    ],
)
def _den_sc(raw, i0, mxs, ex_out, dp_out,
            mxall_v, idx0_v, raw_v, ex_v, den_v, tmp_v, acc_v, shr, sem):
    cid = lax.axis_index("c")
    sid = lax.axis_index("s")
    wid = sid * 2 + cid
    ebase = wid * EPW
    pltpu.sync_copy(mxs, mxall_v)

    def mstep(i, m):
        return jnp.maximum(m, mxall_v[pl.ds(i * L, L)])

    m16 = lax.fori_loop(0, NW, mstep, jnp.full((L,), -1e30, jnp.float32))
    gv = jnp.full((L,), jnp.max(m16))

    def zstep(i, t):
        den_v[pl.ds(i * L, L)] = jnp.zeros((L,), jnp.float32)
        return t

    lax.fori_loop(0, NP // L, zstep, 0)

    def chunk(c, t):
        base = pl.multiple_of(ebase + c * C, 8)
        pltpu.sync_copy(i0.at[pl.ds(base, C)], idx0_v)
        pltpu.sync_copy(raw.at[pl.ds(base, C)], raw_v)
        for g in range(C // L):
            sl = pl.ds(g * L, L)
            e = jnp.exp(raw_v[sl] - gv)
            ex_v[sl] = e
            plsc.addupdate_scatter(den_v, [idx0_v[sl]], e)
        pltpu.sync_copy(ex_v, ex_out.at[pl.ds(base, C)])
        return t

    lax.fori_loop(0, NCH, chunk, 0)

    pltpu.sync_copy(den_v, shr.at[sid])
    plsc.subcore_barrier()
    sbase = pl.multiple_of(sid * SL, 8)

    def z2(i, t):
        acc_v[pl.ds(i * L, L)] = jnp.zeros((L,), jnp.float32)
        return t

    lax.fori_loop(0, SL // L, z2, 0)
    for r in range(16):
        pltpu.sync_copy(shr.at[r, pl.ds(sbase, SL)], tmp_v)

        def astep(i, t):
            s = pl.ds(i * L, L)
            acc_v[s] = acc_v[s] + tmp_v[s]
            return t

        lax.fori_loop(0, SL // L, astep, 0)
    pltpu.sync_copy(acc_v, dp_out.at[cid, pl.ds(sbase, SL)])


@functools.partial(
    pl.kernel,
    out_type=jax.ShapeDtypeStruct((E,), jnp.float32),
    mesh=_MESH,
    compiler_params=pltpu.CompilerParams(needs_layout_passes=False, use_tc_tiling_on_sc=False),
    scratch_types=[
        pltpu.VMEM((NP,), jnp.float32),
        pltpu.VMEM((NP,), jnp.float32),
        pltpu.VMEM((C,), jnp.int32),
        pltpu.VMEM((C,), jnp.float32),
        pltpu.VMEM((C,), jnp.float32),
        pltpu.SemaphoreType.DMA,
    ],
)
def _norm_sc(ex, i0, dp, att_out, den_v, tmpn_v, idx0_v, ex_v, att_v, sem):
    cid = lax.axis_index("c")
    sid = lax.axis_index("s")
    wid = sid * 2 + cid
    ebase = wid * EPW
    pltpu.sync_copy(dp.at[0], den_v)
    pltpu.sync_copy(dp.at[1], tmpn_v)

    def rstep(i, t):
        s = pl.ds(i * L, L)
        den_v[s] = 1.0 / (den_v[s] + tmpn_v[s])
        return t

    lax.fori_loop(0, NP // L, rstep, 0)

    def chunk(c, t):
        base = pl.multiple_of(ebase + c * C, 8)
        pltpu.sync_copy(i0.at[pl.ds(base, C)], idx0_v)
        pltpu.sync_copy(ex.at[pl.ds(base, C)], ex_v)
        for g in range(C // L):
            sl = pl.ds(g * L, L)
            r = plsc.load_gather(den_v, [idx0_v[sl]])
            att_v[sl] = ex_v[sl] * r
        pltpu.sync_copy(att_v, att_out.at[pl.ds(base, C)])
        return t

    lax.fori_loop(0, NCH, chunk, 0)


def kernel(from_s, from_v, to_s, to_v, edge_index, from_frame, to_frame,
           from_pos, to_pos, Wfs, Wts, Wfv, Wtv, Wattn):
    s_in = jnp.concatenate([from_s, to_s], axis=0)
    v_in = jnp.concatenate([from_v.reshape(N, 48), to_v.reshape(N, 48)], axis=0)
    f_in = jnp.concatenate([from_frame.reshape(N, 9), to_frame.reshape(N, 9)], axis=0)
    p_in = jnp.concatenate([from_pos, to_pos], axis=0)

    WT = jnp.stack([Wfs.T, Wts.T])
    eye3 = jnp.eye(3, dtype=jnp.float32)
    W1f = jnp.einsum('hv,kj->vkhj', Wfv, eye3).reshape(48, 12)
    W1t = jnp.einsum('hv,kj->vkhj', Wtv, eye3).reshape(48, 12)
    W1 = jnp.stack([W1f, W1t])
    ST = jnp.stack([jnp.asarray(_S_ALL),
                    jnp.pad(jnp.asarray(_T_ALL), ((0, 3), (0, 0)))])
    K3 = jnp.asarray(_K3)
    w = Wattn[0]
    z4 = jnp.zeros((4,), jnp.float32)
    wp = jnp.stack([jnp.concatenate([w[0:16], w[32:44], z4]),
                    jnp.concatenate([w[16:32], w[47:59], z4])])[:, None, :]
    wv = jnp.concatenate([jnp.zeros((1,), jnp.float32), w[44:47], w[59:62],
                          jnp.zeros((9,), jnp.float32)])

    tab = _node_tables(s_in, v_in, f_in, p_in, WT, W1, ST, K3, wp)

    i0 = edge_index[0]
    i1p = edge_index[1] + N

    raw, mxs = _raw_sc(tab, i0, i1p, wv)
    ex, dp = _den_sc(raw, i0, mxs)
    att = _norm_sc(ex, i0, dp)
    return att[:, None]


# speedup vs baseline: 27.8002x; 1.2742x over previous
import functools

import jax
import jax.numpy as jnp
import numpy as np
from jax import lax
from jax.experimental import pallas as pl
from jax.experimental.pallas import tpu as pltpu
import jax.experimental.pallas.tpu_sc as plsc

N = 10000
E = 320000
NP = 10240
NW = 32
EPW = E // NW
C = 80
NCH = EPW // C
L = 16
SL = NP // 16
BN = 400

_S_ALL = np.zeros((12, 36), np.float32)
_T_ALL = np.zeros((9, 36), np.float32)
for _k in range(3):
    for _h in range(4):
        for _j in range(3):
            _S_ALL[3 * _h + _k, 12 * _k + 3 * _h + _j] = 1.0
            _T_ALL[3 * _k + _j, 12 * _k + 3 * _h + _j] = 1.0
_K3 = np.concatenate([np.eye(12, dtype=np.float32)] * 3, axis=0)


def _silu(x):
    return x * (1.0 / (1.0 + jnp.exp(-x)))


def _node_tc(s_ref, v_ref, f_ref, p_ref, WT_ref, W1_ref, ST_ref, K3_ref, wp_ref,
             tab_ref):
    s = s_ref[...]
    hs = jnp.dot(s, WT_ref[0], preferred_element_type=jnp.float32)
    a_s = jnp.sum(_silu(hs) * wp_ref[0, :, 0:16], axis=1, keepdims=True)
    P = jnp.dot(v_ref[...], W1_ref[0], preferred_element_type=jnp.float32)
    F = f_ref[...]
    PS = jnp.dot(P, ST_ref[0], preferred_element_type=jnp.float32)
    FT = jnp.dot(F, ST_ref[1, 0:9, :], preferred_element_type=jnp.float32)
    G = jnp.dot(PS * FT, K3_ref[...], preferred_element_type=jnp.float32)
    a_g = jnp.sum(_silu(G) * wp_ref[0, :, 16:28], axis=1, keepdims=True)
    a = a_s + a_g
    pad = jnp.zeros((BN, 3), jnp.float32)
    tab_ref[...] = jnp.concatenate([a, p_ref[...], F, pad], axis=1)


def _node_tables(s_in, v_in, f_in, p_in, WT, W1, ST, K3, wp):
    nb = (2 * N) // BN
    side = lambda b: b // (N // BN)
    return pl.pallas_call(
        _node_tc,
        grid=(nb,),
        in_specs=[
            pl.BlockSpec((BN, 128), lambda b: (b, 0)),
            pl.BlockSpec((BN, 48), lambda b: (b, 0)),
            pl.BlockSpec((BN, 9), lambda b: (b, 0)),
            pl.BlockSpec((BN, 3), lambda b: (b, 0)),
            pl.BlockSpec((1, 128, 16), lambda b: (side(b), 0, 0)),
            pl.BlockSpec((1, 48, 12), lambda b: (side(b), 0, 0)),
            pl.BlockSpec((2, 12, 36), lambda b: (0, 0, 0)),
            pl.BlockSpec((36, 12), lambda b: (0, 0)),
            pl.BlockSpec((1, 1, 32), lambda b: (side(b), 0, 0)),
        ],
        out_specs=pl.BlockSpec((BN, 16), lambda b: (b, 0)),
        out_shape=jax.ShapeDtypeStruct((2 * N, 16), jnp.float32),
    )(s_in, v_in, f_in, p_in, WT, W1, ST, K3, wp)


_MESH = plsc.VectorSubcoreMesh(core_axis_name="c", subcore_axis_name="s")


@functools.partial(
    pl.kernel,
    out_type=(jax.ShapeDtypeStruct((E,), jnp.float32),
              jax.ShapeDtypeStruct((NW * L,), jnp.float32)),
    mesh=_MESH,
    compiler_params=pltpu.CompilerParams(needs_layout_passes=False, use_tc_tiling_on_sc=False),
    scratch_types=[
        pltpu.VMEM((C,), jnp.int32),
        pltpu.VMEM((C,), jnp.int32),
        pltpu.VMEM((C,), jnp.int32),
        pltpu.VMEM((C,), jnp.int32),
        pltpu.VMEM((C, L), jnp.float32),
        pltpu.VMEM((C, L), jnp.float32),
        pltpu.VMEM((C, L), jnp.float32),
        pltpu.VMEM((C, L), jnp.float32),
        pltpu.VMEM((C,), jnp.float32),
        pltpu.VMEM((L,), jnp.float32),
        pltpu.VMEM((L,), jnp.float32),
        pltpu.VMEM((6 * L,), jnp.float32),
        pltpu.VMEM((7, C), jnp.float32),
        pltpu.SemaphoreType.DMA,
        pltpu.SemaphoreType.DMA,
    ],
)
def _raw_sc(tab, i0, i1p, wv, raw_out, mx_out,
            ia0, ia1, ib0, ib1, fra, tra, frb, trb,
            raw_v, wv_v, mx_v, ws_v, arg_v, sg0, sg1):
    cid = lax.axis_index("c")
    sid = lax.axis_index("s")
    wid = sid * 2 + cid
    ebase = wid * EPW
    pltpu.sync_copy(wv, wv_v)
    for k in range(6):
        ws_v[pl.ds(k * L, L)] = plsc.load_gather(wv_v, [jnp.full((L,), k + 1, jnp.int32)])

    def fetch_idx(c, d0, d1):
        base = pl.multiple_of(ebase + c * C, 8)
        pltpu.sync_copy(i0.at[pl.ds(base, C)], d0)
        pltpu.sync_copy(i1p.at[pl.ds(base, C)], d1)

    def issue(d0, d1, fr, tr, sg):
        pltpu.async_copy(tab.at[d0], fr, sg)
        pltpu.async_copy(tab.at[d1], tr, sg)

    def drain(d0, d1, fr, tr, sg):
        pltpu.make_async_copy(tab.at[d0], fr, sg).wait()
        pltpu.make_async_copy(tab.at[d1], tr, sg).wait()

    def compute(c, mx, fr, tr):
        base = pl.multiple_of(ebase + c * C, 8)
        for g in range(C // L):
            rows = lax.iota(jnp.int32, L) + g * L
            sl = pl.ds(g * L, L)

            def colf(j):
                return plsc.load_gather(fr, [rows, jnp.full((L,), j, jnp.int32)])

            def colt(j):
                return plsc.load_gather(tr, [rows, jnp.full((L,), j, jnp.int32)])

            arg_v[6, sl] = colf(0) + colt(0)
            dx = colt(1) - colf(1)
            dy = colt(2) - colf(2)
            dz = colt(3) - colf(3)
            for j in range(3):
                arg_v[j, sl] = dx * colf(4 + j) + dy * colf(7 + j) + dz * colf(10 + j)
            for j in range(3):
                arg_v[3 + j, sl] = -(dx * colt(4 + j) + dy * colt(7 + j) + dz * colt(10 + j))
        for g in range(C // L):
            sl = pl.ds(g * L, L)
            acc = arg_v[6, sl]
            for j in range(6):
                cj = arg_v[j, sl]
                acc = acc + ws_v[pl.ds(j * L, L)] * (cj * (1.0 / (1.0 + jnp.exp(-cj))))
            raw_v[sl] = acc
            mx = jnp.maximum(mx, acc)
        pltpu.sync_copy(raw_v, raw_out.at[pl.ds(base, C)])
        return mx

    fetch_idx(0, ia0, ia1)
    issue(ia0, ia1, fra, tra, sg0)

    def pair(t, mx):
        a = 2 * t
        fetch_idx(a + 1, ib0, ib1)
        issue(ib0, ib1, frb, trb, sg1)
        drain(ia0, ia1, fra, tra, sg0)
        mx = compute(a, mx, fra, tra)
        fetch_idx(a + 2, ia0, ia1)
        issue(ia0, ia1, fra, tra, sg0)
        drain(ib0, ib1, frb, trb, sg1)
        return compute(a + 1, mx, frb, trb)

    mx = lax.fori_loop(0, (NCH - 1) // 2, pair, jnp.full((L,), -1e30, jnp.float32))
    drain(ia0, ia1, fra, tra, sg0)
    mx = compute(NCH - 1, mx, fra, tra)
    mx_v[...] = mx
    pltpu.sync_copy(mx_v, mx_out.at[pl.ds(wid * L, L)])


@functools.partial(
    pl.kernel,
    out_type=(jax.ShapeDtypeStruct((E,), jnp.float32),
              jax.ShapeDtypeStruct((2, NP), jnp.float32)),
    mesh=_MESH,
    compiler_params=pltpu.CompilerParams(needs_layout_passes=False, use_tc_tiling_on_sc=False),
    scratch_types=[
        pltpu.VMEM((NW * L,), jnp.float32),
        pltpu.VMEM((C,), jnp.int32),
        pltpu.VMEM((C,), jnp.float32),
        pltpu.VMEM((C,), jnp.float32),
        pltpu.VMEM((NP,), jnp.float32),
        pltpu.VMEM((SL,), jnp.float32),
        pltpu.VMEM((SL,), jnp.float32),
        pltpu.VMEM_SHARED((16, NP), jnp.float32),
        pltpu.SemaphoreType.DMA,
    ],
)
def _den_sc(raw, i0, mxs, ex_out, dp_out,
            mxall_v, idx0_v, raw_v, ex_v, den_v, tmp_v, acc_v, shr, sem):
    cid = lax.axis_index("c")
    sid = lax.axis_index("s")
    wid = sid * 2 + cid
    ebase = wid * EPW
    pltpu.sync_copy(mxs, mxall_v)

    def mstep(i, m):
        return jnp.maximum(m, mxall_v[pl.ds(i * L, L)])

    m16 = lax.fori_loop(0, NW, mstep, jnp.full((L,), -1e30, jnp.float32))
    gv = jnp.full((L,), jnp.max(m16))

    def zstep(i, t):
        den_v[pl.ds(i * L, L)] = jnp.zeros((L,), jnp.float32)
        return t

    lax.fori_loop(0, NP // L, zstep, 0)

    def chunk(c, t):
        base = pl.multiple_of(ebase + c * C, 8)
        pltpu.sync_copy(i0.at[pl.ds(base, C)], idx0_v)
        pltpu.sync_copy(raw.at[pl.ds(base, C)], raw_v)
        for g in range(C // L):
            sl = pl.ds(g * L, L)
            e = jnp.exp(raw_v[sl] - gv)
            ex_v[sl] = e
            plsc.addupdate_scatter(den_v, [idx0_v[sl]], e)
        pltpu.sync_copy(ex_v, ex_out.at[pl.ds(base, C)])
        return t

    lax.fori_loop(0, NCH, chunk, 0)

    pltpu.sync_copy(den_v, shr.at[sid])
    plsc.subcore_barrier()
    sbase = pl.multiple_of(sid * SL, 8)

    def z2(i, t):
        acc_v[pl.ds(i * L, L)] = jnp.zeros((L,), jnp.float32)
        return t

    lax.fori_loop(0, SL // L, z2, 0)
    for r in range(16):
        pltpu.sync_copy(shr.at[r, pl.ds(sbase, SL)], tmp_v)

        def astep(i, t):
            s = pl.ds(i * L, L)
            acc_v[s] = acc_v[s] + tmp_v[s]
            return t

        lax.fori_loop(0, SL // L, astep, 0)
    pltpu.sync_copy(acc_v, dp_out.at[cid, pl.ds(sbase, SL)])


@functools.partial(
    pl.kernel,
    out_type=jax.ShapeDtypeStruct((E,), jnp.float32),
    mesh=_MESH,
    compiler_params=pltpu.CompilerParams(needs_layout_passes=False, use_tc_tiling_on_sc=False),
    scratch_types=[
        pltpu.VMEM((NP,), jnp.float32),
        pltpu.VMEM((NP,), jnp.float32),
        pltpu.VMEM((C,), jnp.int32),
        pltpu.VMEM((C,), jnp.float32),
        pltpu.VMEM((C,), jnp.float32),
        pltpu.SemaphoreType.DMA,
    ],
)
def _norm_sc(ex, i0, dp, att_out, den_v, tmpn_v, idx0_v, ex_v, att_v, sem):
    cid = lax.axis_index("c")
    sid = lax.axis_index("s")
    wid = sid * 2 + cid
    ebase = wid * EPW
    pltpu.sync_copy(dp.at[0], den_v)
    pltpu.sync_copy(dp.at[1], tmpn_v)

    def rstep(i, t):
        s = pl.ds(i * L, L)
        den_v[s] = 1.0 / (den_v[s] + tmpn_v[s])
        return t

    lax.fori_loop(0, NP // L, rstep, 0)

    def chunk(c, t):
        base = pl.multiple_of(ebase + c * C, 8)
        pltpu.sync_copy(i0.at[pl.ds(base, C)], idx0_v)
        pltpu.sync_copy(ex.at[pl.ds(base, C)], ex_v)
        for g in range(C // L):
            sl = pl.ds(g * L, L)
            r = plsc.load_gather(den_v, [idx0_v[sl]])
            att_v[sl] = ex_v[sl] * r
        pltpu.sync_copy(att_v, att_out.at[pl.ds(base, C)])
        return t

    lax.fori_loop(0, NCH, chunk, 0)


def kernel(from_s, from_v, to_s, to_v, edge_index, from_frame, to_frame,
           from_pos, to_pos, Wfs, Wts, Wfv, Wtv, Wattn):
    s_in = jnp.concatenate([from_s, to_s], axis=0)
    v_in = jnp.concatenate([from_v.reshape(N, 48), to_v.reshape(N, 48)], axis=0)
    f_in = jnp.concatenate([from_frame.reshape(N, 9), to_frame.reshape(N, 9)], axis=0)
    p_in = jnp.concatenate([from_pos, to_pos], axis=0)

    WT = jnp.stack([Wfs.T, Wts.T])
    eye3 = jnp.eye(3, dtype=jnp.float32)
    W1f = jnp.einsum('hv,kj->vkhj', Wfv, eye3).reshape(48, 12)
    W1t = jnp.einsum('hv,kj->vkhj', Wtv, eye3).reshape(48, 12)
    W1 = jnp.stack([W1f, W1t])
    ST = jnp.stack([jnp.asarray(_S_ALL),
                    jnp.pad(jnp.asarray(_T_ALL), ((0, 3), (0, 0)))])
    K3 = jnp.asarray(_K3)
    w = Wattn[0]
    z4 = jnp.zeros((4,), jnp.float32)
    wp = jnp.stack([jnp.concatenate([w[0:16], w[32:44], z4]),
                    jnp.concatenate([w[16:32], w[47:59], z4])])[:, None, :]
    wv = jnp.concatenate([jnp.zeros((1,), jnp.float32), w[44:47], w[59:62],
                          jnp.zeros((9,), jnp.float32)])

    tab = _node_tables(s_in, v_in, f_in, p_in, WT, W1, ST, K3, wp)

    i0 = edge_index[0]
    i1p = edge_index[1] + N

    raw, mxs = _raw_sc(tab, i0, i1p, wv)
    ex, dp = _den_sc(raw, i0, mxs)
    att = _norm_sc(ex, i0, dp)
    return att[:, None]


# speedup vs baseline: 38.7684x; 1.3945x over previous
import functools

import jax
import jax.numpy as jnp
import numpy as np
from jax import lax
from jax.experimental import pallas as pl
from jax.experimental.pallas import tpu as pltpu
import jax.experimental.pallas.tpu_sc as plsc

N = 10000
E = 320000
NP = 10240
NW = 32
EPW = E // NW
C = 80
NCH = EPW // C
L = 16
SL = NP // 16
BN = 400

_S_ALL = np.zeros((12, 36), np.float32)
_T_ALL = np.zeros((9, 36), np.float32)
for _k in range(3):
    for _h in range(4):
        for _j in range(3):
            _S_ALL[3 * _h + _k, 12 * _k + 3 * _h + _j] = 1.0
            _T_ALL[3 * _k + _j, 12 * _k + 3 * _h + _j] = 1.0
_K3 = np.concatenate([np.eye(12, dtype=np.float32)] * 3, axis=0)


def _silu(x):
    return x * (1.0 / (1.0 + jnp.exp(-x)))


def _node_tc(s_ref, v_ref, f_ref, p_ref, WT_ref, W1_ref, ST_ref, K3_ref, wp_ref,
             tab_ref):
    s = s_ref[...]
    hs = jnp.dot(s, WT_ref[0], preferred_element_type=jnp.float32)
    a_s = jnp.sum(_silu(hs) * wp_ref[0, :, 0:16], axis=1, keepdims=True)
    P = jnp.dot(v_ref[...], W1_ref[0], preferred_element_type=jnp.float32)
    F = f_ref[...]
    PS = jnp.dot(P, ST_ref[0], preferred_element_type=jnp.float32)
    FT = jnp.dot(F, ST_ref[1, 0:9, :], preferred_element_type=jnp.float32)
    G = jnp.dot(PS * FT, K3_ref[...], preferred_element_type=jnp.float32)
    a_g = jnp.sum(_silu(G) * wp_ref[0, :, 16:28], axis=1, keepdims=True)
    a = a_s + a_g
    pad = jnp.zeros((BN, 3), jnp.float32)
    tab_ref[...] = jnp.concatenate([a, p_ref[...], F, pad], axis=1)


def _node_tables(s_in, v_in, f_in, p_in, WT, W1, ST, K3, wp):
    nb = (2 * N) // BN
    side = lambda b: b // (N // BN)
    return pl.pallas_call(
        _node_tc,
        grid=(nb,),
        in_specs=[
            pl.BlockSpec((BN, 128), lambda b: (b, 0)),
            pl.BlockSpec((BN, 48), lambda b: (b, 0)),
            pl.BlockSpec((BN, 9), lambda b: (b, 0)),
            pl.BlockSpec((BN, 3), lambda b: (b, 0)),
            pl.BlockSpec((1, 128, 16), lambda b: (side(b), 0, 0)),
            pl.BlockSpec((1, 48, 12), lambda b: (side(b), 0, 0)),
            pl.BlockSpec((2, 12, 36), lambda b: (0, 0, 0)),
            pl.BlockSpec((36, 12), lambda b: (0, 0)),
            pl.BlockSpec((1, 1, 32), lambda b: (side(b), 0, 0)),
        ],
        out_specs=pl.BlockSpec((BN, 16), lambda b: (b, 0)),
        out_shape=jax.ShapeDtypeStruct((2 * N, 16), jnp.float32),
    )(s_in, v_in, f_in, p_in, WT, W1, ST, K3, wp)


_MESH = plsc.VectorSubcoreMesh(core_axis_name="c", subcore_axis_name="s")


@functools.partial(
    pl.kernel,
    out_type=(jax.ShapeDtypeStruct((E,), jnp.float32),
              jax.ShapeDtypeStruct((NW * L,), jnp.float32)),
    mesh=_MESH,
    compiler_params=pltpu.CompilerParams(needs_layout_passes=False, use_tc_tiling_on_sc=False),
    scratch_types=[
        pltpu.VMEM((C,), jnp.int32),
        pltpu.VMEM((C,), jnp.int32),
        pltpu.VMEM((C,), jnp.int32),
        pltpu.VMEM((C,), jnp.int32),
        pltpu.VMEM((C, L), jnp.float32),
        pltpu.VMEM((C, L), jnp.float32),
        pltpu.VMEM((C, L), jnp.float32),
        pltpu.VMEM((C, L), jnp.float32),
        pltpu.VMEM((C,), jnp.float32),
        pltpu.VMEM((L,), jnp.float32),
        pltpu.VMEM((L,), jnp.float32),
        pltpu.VMEM((6 * L,), jnp.float32),
        pltpu.VMEM((7, C), jnp.float32),
        pltpu.SemaphoreType.DMA,
        pltpu.SemaphoreType.DMA,
    ],
)
def _raw_sc(tab, i0, i1p, wv, raw_out, mx_out,
            ia0, ia1, ib0, ib1, fra, tra, frb, trb,
            raw_v, wv_v, mx_v, ws_v, arg_v, sg0, sg1):
    cid = lax.axis_index("c")
    sid = lax.axis_index("s")
    wid = sid * 2 + cid
    ebase = wid * EPW
    pltpu.sync_copy(wv, wv_v)
    for k in range(6):
        ws_v[pl.ds(k * L, L)] = plsc.load_gather(wv_v, [jnp.full((L,), k + 1, jnp.int32)])

    def fetch_idx(c, d0, d1):
        base = pl.multiple_of(ebase + c * C, 8)
        pltpu.sync_copy(i0.at[pl.ds(base, C)], d0)
        pltpu.sync_copy(i1p.at[pl.ds(base, C)], d1)

    def issue(d0, d1, fr, tr, sg):
        pltpu.async_copy(tab.at[d0], fr, sg)
        pltpu.async_copy(tab.at[d1], tr, sg)

    def drain(d0, d1, fr, tr, sg):
        pltpu.make_async_copy(tab.at[d0], fr, sg).wait()
        pltpu.make_async_copy(tab.at[d1], tr, sg).wait()

    def compute(c, mx, fr, tr):
        base = pl.multiple_of(ebase + c * C, 8)
        for g in range(C // L):
            rows = lax.iota(jnp.int32, L) + g * L
            sl = pl.ds(g * L, L)

            def colf(j):
                return plsc.load_gather(fr, [rows, jnp.full((L,), j, jnp.int32)])

            def colt(j):
                return plsc.load_gather(tr, [rows, jnp.full((L,), j, jnp.int32)])

            arg_v[6, sl] = colf(0) + colt(0)
            dx = colt(1) - colf(1)
            dy = colt(2) - colf(2)
            dz = colt(3) - colf(3)
            for j in range(3):
                arg_v[j, sl] = dx * colf(4 + j) + dy * colf(7 + j) + dz * colf(10 + j)
            for j in range(3):
                arg_v[3 + j, sl] = -(dx * colt(4 + j) + dy * colt(7 + j) + dz * colt(10 + j))
        for g in range(C // L):
            sl = pl.ds(g * L, L)
            acc = arg_v[6, sl]
            for j in range(6):
                cj = arg_v[j, sl]
                acc = acc + ws_v[pl.ds(j * L, L)] * (cj * (1.0 / (1.0 + jnp.exp(-cj))))
            raw_v[sl] = acc
            mx = jnp.maximum(mx, acc)
        pltpu.sync_copy(raw_v, raw_out.at[pl.ds(base, C)])
        return mx

    fetch_idx(0, ia0, ia1)
    issue(ia0, ia1, fra, tra, sg0)

    def pair(t, mx):
        a = 2 * t
        fetch_idx(a + 1, ib0, ib1)
        issue(ib0, ib1, frb, trb, sg1)
        drain(ia0, ia1, fra, tra, sg0)
        mx = compute(a, mx, fra, tra)
        fetch_idx(a + 2, ia0, ia1)
        issue(ia0, ia1, fra, tra, sg0)
        drain(ib0, ib1, frb, trb, sg1)
        return compute(a + 1, mx, frb, trb)

    mx = lax.fori_loop(0, (NCH - 1) // 2, pair, jnp.full((L,), -1e30, jnp.float32))
    drain(ia0, ia1, fra, tra, sg0)
    mx = compute(NCH - 1, mx, fra, tra)
    mx_v[...] = mx
    pltpu.sync_copy(mx_v, mx_out.at[pl.ds(wid * L, L)])


@functools.partial(
    pl.kernel,
    out_type=(jax.ShapeDtypeStruct((E,), jnp.float32),
              jax.ShapeDtypeStruct((2, NP), jnp.float32)),
    mesh=_MESH,
    compiler_params=pltpu.CompilerParams(needs_layout_passes=False, use_tc_tiling_on_sc=False),
    scratch_types=[
        pltpu.VMEM((NW * L,), jnp.float32),
        pltpu.VMEM((C,), jnp.int32),
        pltpu.VMEM((C,), jnp.int32),
        pltpu.VMEM((C,), jnp.float32),
        pltpu.VMEM((C,), jnp.float32),
        pltpu.VMEM((C,), jnp.float32),
        pltpu.VMEM((NP,), jnp.float32),
        pltpu.VMEM((SL,), jnp.float32),
        pltpu.VMEM((SL,), jnp.float32),
        pltpu.VMEM_SHARED((16, NP), jnp.float32),
        pltpu.SemaphoreType.DMA,
        pltpu.SemaphoreType.DMA,
    ],
)
def _den_sc(raw, i0, mxs, ex_out, dp_out,
            mxall_v, iA, iB, rA, rB, ex_v, den_v, tmp_v, acc_v, shr, sA, sB):
    cid = lax.axis_index("c")
    sid = lax.axis_index("s")
    wid = sid * 2 + cid
    ebase = wid * EPW
    pltpu.sync_copy(mxs, mxall_v)

    def mstep(i, m):
        return jnp.maximum(m, mxall_v[pl.ds(i * L, L)])

    m16 = lax.fori_loop(0, NW, mstep, jnp.full((L,), -1e30, jnp.float32))
    gv = jnp.full((L,), jnp.max(m16))

    def zstep(i, t):
        den_v[pl.ds(i * L, L)] = jnp.zeros((L,), jnp.float32)
        return t

    lax.fori_loop(0, NP // L, zstep, 0)

    def loads(c, di, dr, sg):
        base = pl.multiple_of(ebase + c * C, 8)
        pltpu.async_copy(i0.at[pl.ds(base, C)], di, sg)
        pltpu.async_copy(raw.at[pl.ds(base, C)], dr, sg)

    def drain(di, dr, sg):
        pltpu.make_async_copy(i0.at[pl.ds(0, C)], di, sg).wait()
        pltpu.make_async_copy(raw.at[pl.ds(0, C)], dr, sg).wait()

    def compute(c, di, dr):
        base = pl.multiple_of(ebase + c * C, 8)
        for g in range(C // L):
            sl = pl.ds(g * L, L)
            e = jnp.exp(dr[sl] - gv)
            ex_v[sl] = e
            plsc.addupdate_scatter(den_v, [di[sl]], e)
        pltpu.sync_copy(ex_v, ex_out.at[pl.ds(base, C)])

    loads(0, iA, rA, sA)

    def pair(t, u):
        a = 2 * t
        loads(a + 1, iB, rB, sB)
        drain(iA, rA, sA)
        compute(a, iA, rA)
        loads(a + 2, iA, rA, sA)
        drain(iB, rB, sB)
        compute(a + 1, iB, rB)
        return u

    lax.fori_loop(0, (NCH - 1) // 2, pair, 0)
    drain(iA, rA, sA)
    compute(NCH - 1, iA, rA)

    pltpu.sync_copy(den_v, shr.at[sid])
    plsc.subcore_barrier()
    sbase = pl.multiple_of(sid * SL, 8)

    def z2(i, t):
        acc_v[pl.ds(i * L, L)] = jnp.zeros((L,), jnp.float32)
        return t

    lax.fori_loop(0, SL // L, z2, 0)
    for r in range(16):
        pltpu.sync_copy(shr.at[r, pl.ds(sbase, SL)], tmp_v)

        def astep(i, t):
            s = pl.ds(i * L, L)
            acc_v[s] = acc_v[s] + tmp_v[s]
            return t

        lax.fori_loop(0, SL // L, astep, 0)
    pltpu.sync_copy(acc_v, dp_out.at[cid, pl.ds(sbase, SL)])


@functools.partial(
    pl.kernel,
    out_type=jax.ShapeDtypeStruct((E,), jnp.float32),
    mesh=_MESH,
    compiler_params=pltpu.CompilerParams(needs_layout_passes=False, use_tc_tiling_on_sc=False),
    scratch_types=[
        pltpu.VMEM((NP,), jnp.float32),
        pltpu.VMEM((NP,), jnp.float32),
        pltpu.VMEM((C,), jnp.int32),
        pltpu.VMEM((C,), jnp.int32),
        pltpu.VMEM((C,), jnp.float32),
        pltpu.VMEM((C,), jnp.float32),
        pltpu.VMEM((C,), jnp.float32),
        pltpu.SemaphoreType.DMA,
        pltpu.SemaphoreType.DMA,
    ],
)
def _norm_sc(ex, i0, dp, att_out, den_v, tmpn_v, iA, iB, eA, eB, att_v, sA, sB):
    cid = lax.axis_index("c")
    sid = lax.axis_index("s")
    wid = sid * 2 + cid
    ebase = wid * EPW
    pltpu.sync_copy(dp.at[0], den_v)
    pltpu.sync_copy(dp.at[1], tmpn_v)

    def rstep(i, t):
        s = pl.ds(i * L, L)
        den_v[s] = 1.0 / (den_v[s] + tmpn_v[s])
        return t

    lax.fori_loop(0, NP // L, rstep, 0)

    def loads(c, di, de, sg):
        base = pl.multiple_of(ebase + c * C, 8)
        pltpu.async_copy(i0.at[pl.ds(base, C)], di, sg)
        pltpu.async_copy(ex.at[pl.ds(base, C)], de, sg)

    def drain(di, de, sg):
        pltpu.make_async_copy(i0.at[pl.ds(0, C)], di, sg).wait()
        pltpu.make_async_copy(ex.at[pl.ds(0, C)], de, sg).wait()

    def compute(c, di, de):
        base = pl.multiple_of(ebase + c * C, 8)
        for g in range(C // L):
            sl = pl.ds(g * L, L)
            r = plsc.load_gather(den_v, [di[sl]])
            att_v[sl] = de[sl] * r
        pltpu.sync_copy(att_v, att_out.at[pl.ds(base, C)])

    loads(0, iA, eA, sA)

    def pair(t, u):
        a = 2 * t
        loads(a + 1, iB, eB, sB)
        drain(iA, eA, sA)
        compute(a, iA, eA)
        loads(a + 2, iA, eA, sA)
        drain(iB, eB, sB)
        compute(a + 1, iB, eB)
        return u

    lax.fori_loop(0, (NCH - 1) // 2, pair, 0)
    drain(iA, eA, sA)
    compute(NCH - 1, iA, eA)


def kernel(from_s, from_v, to_s, to_v, edge_index, from_frame, to_frame,
           from_pos, to_pos, Wfs, Wts, Wfv, Wtv, Wattn):
    s_in = jnp.concatenate([from_s, to_s], axis=0)
    v_in = jnp.concatenate([from_v.reshape(N, 48), to_v.reshape(N, 48)], axis=0)
    f_in = jnp.concatenate([from_frame.reshape(N, 9), to_frame.reshape(N, 9)], axis=0)
    p_in = jnp.concatenate([from_pos, to_pos], axis=0)

    WT = jnp.stack([Wfs.T, Wts.T])
    eye3 = jnp.eye(3, dtype=jnp.float32)
    W1f = jnp.einsum('hv,kj->vkhj', Wfv, eye3).reshape(48, 12)
    W1t = jnp.einsum('hv,kj->vkhj', Wtv, eye3).reshape(48, 12)
    W1 = jnp.stack([W1f, W1t])
    ST = jnp.stack([jnp.asarray(_S_ALL),
                    jnp.pad(jnp.asarray(_T_ALL), ((0, 3), (0, 0)))])
    K3 = jnp.asarray(_K3)
    w = Wattn[0]
    z4 = jnp.zeros((4,), jnp.float32)
    wp = jnp.stack([jnp.concatenate([w[0:16], w[32:44], z4]),
                    jnp.concatenate([w[16:32], w[47:59], z4])])[:, None, :]
    wv = jnp.concatenate([jnp.zeros((1,), jnp.float32), w[44:47], w[59:62],
                          jnp.zeros((9,), jnp.float32)])

    tab = _node_tables(s_in, v_in, f_in, p_in, WT, W1, ST, K3, wp)

    i0 = edge_index[0]
    i1p = edge_index[1] + N

    raw, mxs = _raw_sc(tab, i0, i1p, wv)
    ex, dp = _den_sc(raw, i0, mxs)
    att = _norm_sc(ex, i0, dp)
    return att[:, None]


# speedup vs baseline: 39.8109x; 1.0269x over previous
import functools

import jax
import jax.numpy as jnp
import numpy as np
from jax import lax
from jax.experimental import pallas as pl
from jax.experimental.pallas import tpu as pltpu
import jax.experimental.pallas.tpu_sc as plsc

N = 10000
E = 320000
NP = 10240
NW = 32
EPW = E // NW
C = 80
NCH = EPW // C
L = 16
SL = NP // 16
BN = 400

_S_ALL = np.zeros((12, 36), np.float32)
_T_ALL = np.zeros((9, 36), np.float32)
for _k in range(3):
    for _h in range(4):
        for _j in range(3):
            _S_ALL[3 * _h + _k, 12 * _k + 3 * _h + _j] = 1.0
            _T_ALL[3 * _k + _j, 12 * _k + 3 * _h + _j] = 1.0
_K3 = np.concatenate([np.eye(12, dtype=np.float32)] * 3, axis=0)


def _silu(x):
    return x * (1.0 / (1.0 + jnp.exp(-x)))


def _node_tc(s_ref, v_ref, f_ref, p_ref, WT_ref, W1_ref, ST_ref, K3_ref, wp_ref,
             tab_ref):
    s = s_ref[...]
    hs = jnp.dot(s, WT_ref[0], preferred_element_type=jnp.float32)
    a_s = jnp.sum(_silu(hs) * wp_ref[0, :, 0:16], axis=1, keepdims=True)
    P = jnp.dot(v_ref[...], W1_ref[0], preferred_element_type=jnp.float32)
    F = f_ref[...]
    PS = jnp.dot(P, ST_ref[0], preferred_element_type=jnp.float32)
    FT = jnp.dot(F, ST_ref[1, 0:9, :], preferred_element_type=jnp.float32)
    G = jnp.dot(PS * FT, K3_ref[...], preferred_element_type=jnp.float32)
    a_g = jnp.sum(_silu(G) * wp_ref[0, :, 16:28], axis=1, keepdims=True)
    a = a_s + a_g
    pad = jnp.zeros((BN, 3), jnp.float32)
    tab_ref[...] = jnp.concatenate([a, p_ref[...], F, pad], axis=1)


def _node_tables(s_in, v_in, f_in, p_in, WT, W1, ST, K3, wp):
    nb = (2 * N) // BN
    side = lambda b: b // (N // BN)
    return pl.pallas_call(
        _node_tc,
        grid=(nb,),
        in_specs=[
            pl.BlockSpec((BN, 128), lambda b: (b, 0)),
            pl.BlockSpec((BN, 48), lambda b: (b, 0)),
            pl.BlockSpec((BN, 9), lambda b: (b, 0)),
            pl.BlockSpec((BN, 3), lambda b: (b, 0)),
            pl.BlockSpec((1, 128, 16), lambda b: (side(b), 0, 0)),
            pl.BlockSpec((1, 48, 12), lambda b: (side(b), 0, 0)),
            pl.BlockSpec((2, 12, 36), lambda b: (0, 0, 0)),
            pl.BlockSpec((36, 12), lambda b: (0, 0)),
            pl.BlockSpec((1, 1, 32), lambda b: (side(b), 0, 0)),
        ],
        out_specs=pl.BlockSpec((BN, 16), lambda b: (b, 0)),
        out_shape=jax.ShapeDtypeStruct((2 * N, 16), jnp.float32),
    )(s_in, v_in, f_in, p_in, WT, W1, ST, K3, wp)


_MESH = plsc.VectorSubcoreMesh(core_axis_name="c", subcore_axis_name="s")


@functools.partial(
    pl.kernel,
    out_type=(jax.ShapeDtypeStruct((E,), jnp.float32),
              jax.ShapeDtypeStruct((NW * L,), jnp.float32)),
    mesh=_MESH,
    compiler_params=pltpu.CompilerParams(needs_layout_passes=False, use_tc_tiling_on_sc=False),
    scratch_types=[
        pltpu.VMEM((C,), jnp.int32),
        pltpu.VMEM((C,), jnp.int32),
        pltpu.VMEM((C,), jnp.int32),
        pltpu.VMEM((C,), jnp.int32),
        pltpu.VMEM((C, L), jnp.float32),
        pltpu.VMEM((C, L), jnp.float32),
        pltpu.VMEM((C, L), jnp.float32),
        pltpu.VMEM((C, L), jnp.float32),
        pltpu.VMEM((C,), jnp.float32),
        pltpu.VMEM((C,), jnp.float32),
        pltpu.VMEM((L,), jnp.float32),
        pltpu.VMEM((L,), jnp.float32),
        pltpu.VMEM((6 * L,), jnp.float32),
        pltpu.VMEM((7, C), jnp.float32),
        pltpu.SemaphoreType.DMA,
        pltpu.SemaphoreType.DMA,
        pltpu.SemaphoreType.DMA,
        pltpu.SemaphoreType.DMA,
    ],
)
def _raw_sc(tab, i0, i1p, wv, raw_out, mx_out,
            ia0, ia1, ib0, ib1, fra, tra, frb, trb,
            raw_a, raw_b, wv_v, mx_v, ws_v, arg_v, sg0, sg1, swA, swB):
    cid = lax.axis_index("c")
    sid = lax.axis_index("s")
    wid = sid * 2 + cid
    ebase = wid * EPW
    pltpu.sync_copy(wv, wv_v)
    for k in range(6):
        ws_v[pl.ds(k * L, L)] = plsc.load_gather(wv_v, [jnp.full((L,), k + 1, jnp.int32)])

    def fetch_idx(c, d0, d1):
        base = pl.multiple_of(ebase + c * C, 8)
        pltpu.sync_copy(i0.at[pl.ds(base, C)], d0)
        pltpu.sync_copy(i1p.at[pl.ds(base, C)], d1)

    def issue(d0, d1, fr, tr, sg):
        pltpu.async_copy(tab.at[d0], fr, sg)
        pltpu.async_copy(tab.at[d1], tr, sg)

    def drain(d0, d1, fr, tr, sg):
        pltpu.make_async_copy(tab.at[d0], fr, sg).wait()
        pltpu.make_async_copy(tab.at[d1], tr, sg).wait()

    def compute(c, mx, fr, tr, rbuf, sw):
        base = pl.multiple_of(ebase + c * C, 8)
        for g in range(C // L):
            rows = lax.iota(jnp.int32, L) + g * L
            sl = pl.ds(g * L, L)

            def colf(j):
                return plsc.load_gather(fr, [rows, jnp.full((L,), j, jnp.int32)])

            def colt(j):
                return plsc.load_gather(tr, [rows, jnp.full((L,), j, jnp.int32)])

            arg_v[6, sl] = colf(0) + colt(0)
            dx = colt(1) - colf(1)
            dy = colt(2) - colf(2)
            dz = colt(3) - colf(3)
            for j in range(3):
                arg_v[j, sl] = dx * colf(4 + j) + dy * colf(7 + j) + dz * colf(10 + j)
            for j in range(3):
                arg_v[3 + j, sl] = -(dx * colt(4 + j) + dy * colt(7 + j) + dz * colt(10 + j))
        for g in range(C // L):
            sl = pl.ds(g * L, L)
            acc = arg_v[6, sl]
            for j in range(6):
                cj = arg_v[j, sl]
                acc = acc + ws_v[pl.ds(j * L, L)] * (cj * (1.0 / (1.0 + jnp.exp(-cj))))
            rbuf[sl] = acc
            mx = jnp.maximum(mx, acc)
        pltpu.async_copy(rbuf, raw_out.at[pl.ds(base, C)], sw)
        return mx

    def drain_w(rbuf, sw):
        pltpu.make_async_copy(rbuf, raw_out.at[pl.ds(0, C)], sw).wait()

    fetch_idx(0, ia0, ia1)
    issue(ia0, ia1, fra, tra, sg0)
    mx = jnp.full((L,), -1e30, jnp.float32)
    fetch_idx(1, ib0, ib1)
    issue(ib0, ib1, frb, trb, sg1)
    drain(ia0, ia1, fra, tra, sg0)
    mx = compute(0, mx, fra, tra, raw_a, swA)
    fetch_idx(2, ia0, ia1)
    issue(ia0, ia1, fra, tra, sg0)
    drain(ib0, ib1, frb, trb, sg1)
    mx = compute(1, mx, frb, trb, raw_b, swB)

    def pair(t, mx):
        a = 2 * t
        fetch_idx(a + 1, ib0, ib1)
        issue(ib0, ib1, frb, trb, sg1)
        drain(ia0, ia1, fra, tra, sg0)
        drain_w(raw_a, swA)
        mx = compute(a, mx, fra, tra, raw_a, swA)
        fetch_idx(a + 2, ia0, ia1)
        issue(ia0, ia1, fra, tra, sg0)
        drain(ib0, ib1, frb, trb, sg1)
        drain_w(raw_b, swB)
        return compute(a + 1, mx, frb, trb, raw_b, swB)

    mx = lax.fori_loop(1, (NCH - 1) // 2, pair, mx)
    drain(ia0, ia1, fra, tra, sg0)
    drain_w(raw_a, swA)
    mx = compute(NCH - 1, mx, fra, tra, raw_a, swA)
    drain_w(raw_a, swA)
    drain_w(raw_b, swB)
    mx_v[...] = mx
    pltpu.sync_copy(mx_v, mx_out.at[pl.ds(wid * L, L)])


@functools.partial(
    pl.kernel,
    out_type=(jax.ShapeDtypeStruct((E,), jnp.float32),
              jax.ShapeDtypeStruct((2, NP), jnp.float32)),
    mesh=_MESH,
    compiler_params=pltpu.CompilerParams(needs_layout_passes=False, use_tc_tiling_on_sc=False),
    scratch_types=[
        pltpu.VMEM((NW * L,), jnp.float32),
        pltpu.VMEM((C,), jnp.int32),
        pltpu.VMEM((C,), jnp.int32),
        pltpu.VMEM((C,), jnp.float32),
        pltpu.VMEM((C,), jnp.float32),
        pltpu.VMEM((C,), jnp.float32),
        pltpu.VMEM((C,), jnp.float32),
        pltpu.VMEM((NP,), jnp.float32),
        pltpu.VMEM((SL,), jnp.float32),
        pltpu.VMEM((SL,), jnp.float32),
        pltpu.VMEM_SHARED((16, NP), jnp.float32),
        pltpu.SemaphoreType.DMA,
        pltpu.SemaphoreType.DMA,
        pltpu.SemaphoreType.DMA,
        pltpu.SemaphoreType.DMA,
    ],
)
def _den_sc(raw, i0, mxs, ex_out, dp_out,
            mxall_v, iA, iB, rA, rB, exA, exB, den_v, tmp_v, acc_v, shr,
            sA, sB, swA, swB):
    cid = lax.axis_index("c")
    sid = lax.axis_index("s")
    wid = sid * 2 + cid
    ebase = wid * EPW
    pltpu.sync_copy(mxs, mxall_v)

    def mstep(i, m):
        return jnp.maximum(m, mxall_v[pl.ds(i * L, L)])

    m16 = lax.fori_loop(0, NW, mstep, jnp.full((L,), -1e30, jnp.float32))
    gv = jnp.full((L,), jnp.max(m16))

    def zstep(i, t):
        den_v[pl.ds(i * L, L)] = jnp.zeros((L,), jnp.float32)
        return t

    lax.fori_loop(0, NP // L, zstep, 0)

    def loads(c, di, dr, sg):
        base = pl.multiple_of(ebase + c * C, 8)
        pltpu.async_copy(i0.at[pl.ds(base, C)], di, sg)
        pltpu.async_copy(raw.at[pl.ds(base, C)], dr, sg)

    def drain(di, dr, sg):
        pltpu.make_async_copy(i0.at[pl.ds(0, C)], di, sg).wait()
        pltpu.make_async_copy(raw.at[pl.ds(0, C)], dr, sg).wait()

    def compute(c, di, dr, ebuf, sw):
        base = pl.multiple_of(ebase + c * C, 8)
        for g in range(C // L):
            sl = pl.ds(g * L, L)
            e = jnp.exp(dr[sl] - gv)
            ebuf[sl] = e
            plsc.addupdate_scatter(den_v, [di[sl]], e)
        pltpu.async_copy(ebuf, ex_out.at[pl.ds(base, C)], sw)

    def drain_w(ebuf, sw):
        pltpu.make_async_copy(ebuf, ex_out.at[pl.ds(0, C)], sw).wait()

    loads(0, iA, rA, sA)
    loads(1, iB, rB, sB)
    drain(iA, rA, sA)
    compute(0, iA, rA, exA, swA)
    loads(2, iA, rA, sA)
    drain(iB, rB, sB)
    compute(1, iB, rB, exB, swB)

    def pair(t, u):
        a = 2 * t
        loads(a + 1, iB, rB, sB)
        drain(iA, rA, sA)
        drain_w(exA, swA)
        compute(a, iA, rA, exA, swA)
        loads(a + 2, iA, rA, sA)
        drain(iB, rB, sB)
        drain_w(exB, swB)
        compute(a + 1, iB, rB, exB, swB)
        return u

    lax.fori_loop(1, (NCH - 1) // 2, pair, 0)
    drain(iA, rA, sA)
    drain_w(exA, swA)
    compute(NCH - 1, iA, rA, exA, swA)
    drain_w(exA, swA)
    drain_w(exB, swB)

    pltpu.sync_copy(den_v, shr.at[sid])
    plsc.subcore_barrier()
    sbase = pl.multiple_of(sid * SL, 8)

    def z2(i, t):
        acc_v[pl.ds(i * L, L)] = jnp.zeros((L,), jnp.float32)
        return t

    lax.fori_loop(0, SL // L, z2, 0)
    for r in range(16):
        pltpu.sync_copy(shr.at[r, pl.ds(sbase, SL)], tmp_v)

        def astep(i, t):
            s = pl.ds(i * L, L)
            acc_v[s] = acc_v[s] + tmp_v[s]
            return t

        lax.fori_loop(0, SL // L, astep, 0)
    pltpu.sync_copy(acc_v, dp_out.at[cid, pl.ds(sbase, SL)])


@functools.partial(
    pl.kernel,
    out_type=jax.ShapeDtypeStruct((E,), jnp.float32),
    mesh=_MESH,
    compiler_params=pltpu.CompilerParams(needs_layout_passes=False, use_tc_tiling_on_sc=False),
    scratch_types=[
        pltpu.VMEM((NP,), jnp.float32),
        pltpu.VMEM((NP,), jnp.float32),
        pltpu.VMEM((C,), jnp.int32),
        pltpu.VMEM((C,), jnp.int32),
        pltpu.VMEM((C,), jnp.float32),
        pltpu.VMEM((C,), jnp.float32),
        pltpu.VMEM((C,), jnp.float32),
        pltpu.VMEM((C,), jnp.float32),
        pltpu.SemaphoreType.DMA,
        pltpu.SemaphoreType.DMA,
        pltpu.SemaphoreType.DMA,
        pltpu.SemaphoreType.DMA,
    ],
)
def _norm_sc(ex, i0, dp, att_out, den_v, tmpn_v, iA, iB, eA, eB, attA, attB,
             sA, sB, swA, swB):
    cid = lax.axis_index("c")
    sid = lax.axis_index("s")
    wid = sid * 2 + cid
    ebase = wid * EPW
    pltpu.sync_copy(dp.at[0], den_v)
    pltpu.sync_copy(dp.at[1], tmpn_v)

    def rstep(i, t):
        s = pl.ds(i * L, L)
        den_v[s] = 1.0 / (den_v[s] + tmpn_v[s])
        return t

    lax.fori_loop(0, NP // L, rstep, 0)

    def loads(c, di, de, sg):
        base = pl.multiple_of(ebase + c * C, 8)
        pltpu.async_copy(i0.at[pl.ds(base, C)], di, sg)
        pltpu.async_copy(ex.at[pl.ds(base, C)], de, sg)

    def drain(di, de, sg):
        pltpu.make_async_copy(i0.at[pl.ds(0, C)], di, sg).wait()
        pltpu.make_async_copy(ex.at[pl.ds(0, C)], de, sg).wait()

    def compute(c, di, de, abuf, sw):
        base = pl.multiple_of(ebase + c * C, 8)
        for g in range(C // L):
            sl = pl.ds(g * L, L)
            r = plsc.load_gather(den_v, [di[sl]])
            abuf[sl] = de[sl] * r
        pltpu.async_copy(abuf, att_out.at[pl.ds(base, C)], sw)

    def drain_w(abuf, sw):
        pltpu.make_async_copy(abuf, att_out.at[pl.ds(0, C)], sw).wait()

    loads(0, iA, eA, sA)
    loads(1, iB, eB, sB)
    drain(iA, eA, sA)
    compute(0, iA, eA, attA, swA)
    loads(2, iA, eA, sA)
    drain(iB, eB, sB)
    compute(1, iB, eB, attB, swB)

    def pair(t, u):
        a = 2 * t
        loads(a + 1, iB, eB, sB)
        drain(iA, eA, sA)
        drain_w(attA, swA)
        compute(a, iA, eA, attA, swA)
        loads(a + 2, iA, eA, sA)
        drain(iB, eB, sB)
        drain_w(attB, swB)
        compute(a + 1, iB, eB, attB, swB)
        return u

    lax.fori_loop(1, (NCH - 1) // 2, pair, 0)
    drain(iA, eA, sA)
    drain_w(attA, swA)
    compute(NCH - 1, iA, eA, attA, swA)
    drain_w(attA, swA)
    drain_w(attB, swB)


def kernel(from_s, from_v, to_s, to_v, edge_index, from_frame, to_frame,
           from_pos, to_pos, Wfs, Wts, Wfv, Wtv, Wattn):
    s_in = jnp.concatenate([from_s, to_s], axis=0)
    v_in = jnp.concatenate([from_v.reshape(N, 48), to_v.reshape(N, 48)], axis=0)
    f_in = jnp.concatenate([from_frame.reshape(N, 9), to_frame.reshape(N, 9)], axis=0)
    p_in = jnp.concatenate([from_pos, to_pos], axis=0)

    WT = jnp.stack([Wfs.T, Wts.T])
    eye3 = jnp.eye(3, dtype=jnp.float32)
    W1f = jnp.einsum('hv,kj->vkhj', Wfv, eye3).reshape(48, 12)
    W1t = jnp.einsum('hv,kj->vkhj', Wtv, eye3).reshape(48, 12)
    W1 = jnp.stack([W1f, W1t])
    ST = jnp.stack([jnp.asarray(_S_ALL),
                    jnp.pad(jnp.asarray(_T_ALL), ((0, 3), (0, 0)))])
    K3 = jnp.asarray(_K3)
    w = Wattn[0]
    z4 = jnp.zeros((4,), jnp.float32)
    wp = jnp.stack([jnp.concatenate([w[0:16], w[32:44], z4]),
                    jnp.concatenate([w[16:32], w[47:59], z4])])[:, None, :]
    wv = jnp.concatenate([jnp.zeros((1,), jnp.float32), w[44:47], w[59:62],
                          jnp.zeros((9,), jnp.float32)])

    tab = _node_tables(s_in, v_in, f_in, p_in, WT, W1, ST, K3, wp)

    i0 = edge_index[0]
    i1p = edge_index[1] + N

    raw, mxs = _raw_sc(tab, i0, i1p, wv)
    ex, dp = _den_sc(raw, i0, mxs)
    att = _norm_sc(ex, i0, dp)
    return att[:, None]
